# Initial kernel scaffold; baseline (speedup 1.0000x reference)
import jax
import jax.numpy as jnp
from jax.experimental import pallas as pl


def kernel(x, edge_index, W1, b1, g1, be1, W2, b2, g2, be2, W3, b3, g3, be3, W4, b4, g4, be4, Wc, bc):
    raise NotImplementedError("write your pallas kernel here")


# speedup vs baseline: 1.0359x; 1.0359x over previous
import jax
import jax.numpy as jnp
from jax.experimental import pallas as pl


def _gcn_conv(x, src, dst, W, b):
    n = x.shape[0]
    h = x @ W
    deg = jnp.zeros((n,), h.dtype).at[dst].add(1.0) + 1.0
    dis = jax.lax.rsqrt(deg)
    norm = dis[src] * dis[dst]
    agg = jnp.zeros_like(h).at[dst].add(norm[:, None] * h[src])
    agg = agg + (dis * dis)[:, None] * h
    return agg + b


def _batch_norm(h, gamma, beta, eps=1e-5):
    mu = jnp.mean(h, axis=0)
    var = jnp.mean((h - mu) ** 2, axis=0)
    return gamma * (h - mu) * jax.lax.rsqrt(var + eps) + beta


def _mm_kernel(x_ref, w_ref, b_ref, o_ref):
    o_ref[...] = jnp.dot(x_ref[...], w_ref[...],
                         preferred_element_type=jnp.float32) + b_ref[...]


def _classifier(h, Wc, bc):
    n, k = h.shape
    nc = Wc.shape[1]
    bm = 1000
    return pl.pallas_call(
        _mm_kernel,
        grid=(n // bm,),
        in_specs=[
            pl.BlockSpec((bm, k), lambda i: (i, 0)),
            pl.BlockSpec((k, nc), lambda i: (0, 0)),
            pl.BlockSpec((1, nc), lambda i: (0, 0)),
        ],
        out_specs=pl.BlockSpec((bm, nc), lambda i: (i, 0)),
        out_shape=jax.ShapeDtypeStruct((n, nc), jnp.float32),
    )(h, Wc, bc.reshape(1, nc))


def kernel(x, edge_index, W1, b1, g1, be1, W2, b2, g2, be2, W3, b3, g3, be3, W4, b4, g4, be4, Wc, bc):
    src, dst = edge_index[0], edge_index[1]
    h = _gcn_conv(x, src, dst, W1, b1)
    h = _batch_norm(jax.nn.relu(h), g1, be1)
    h = _gcn_conv(h, src, dst, W2, b2)
    h = _batch_norm(jax.nn.relu(h), g2, be2)
    h = _gcn_conv(h, src, dst, W3, b3)
    h = _batch_norm(jax.nn.relu(h), g3, be3)
    h = _gcn_conv(h, src, dst, W4, b4)
    h = _batch_norm(jax.nn.relu(h), g4, be4)
    return _classifier(h, Wc, bc)


# speedup vs baseline: 5.5320x; 5.3401x over previous
import functools

import jax
import jax.numpy as jnp
from jax import lax
from jax.experimental import pallas as pl
from jax.experimental.pallas import tpu as pltpu
from jax.experimental.pallas import tpu_sc as plsc

N = 10000
E = 320000
NS = 16
CHUNK = 128
CHC = 158
CHE = 79
E_PAD = NS * CHC * CHUNK
WP = 128
N_ACC = 10240
BM = 400
EPS = 1e-5

_mesh = plsc.VectorSubcoreMesh(core_axis_name="c", subcore_axis_name="s")


def _agg_common(ch, load_idx):
    def body(p_hbm, src_hbm, dst_hbm, out_hbm, srcv, dstv, gbuf, acc, sem):
        c = lax.axis_index("c")
        s = lax.axis_index("s")

        def zb(i, carry):
            gbuf[i // 8, pl.ds((i % 8) * 16, 16)] = jnp.zeros((16,), jnp.float32)
            return carry
        lax.fori_loop(0, CHUNK * 8, zb, None)

        def z(k, carry):
            pltpu.sync_copy(gbuf, acc.at[pl.ds(s * 640 + k * 128, 128)])
            return carry
        lax.fori_loop(0, 5, z, None)
        load_idx(src_hbm, dst_hbm, srcv, dstv, c, s)
        plsc.subcore_barrier()

        def ed(g, carry):
            pltpu.async_copy(p_hbm.at[srcv.at[g]], gbuf, sem).wait()
            pltpu.sync_copy(gbuf, acc.at[dstv.at[g]], add=True)
            return carry
        lax.fori_loop(0, ch, ed, None)
        plsc.subcore_barrier()

        def dr(k, carry):
            base = s * 640 + k * 128
            pltpu.sync_copy(acc.at[pl.ds(base, 128)], gbuf)
            pltpu.sync_copy(gbuf, out_hbm.at[c, pl.ds(base, 128)])
            return carry
        lax.fori_loop(0, 5, dr, None)

    return body


def _load_idx_edge(src_hbm, dst_hbm, srcv, dstv, c, s):
    w = c * NS + s
    pltpu.sync_copy(src_hbm.at[w], srcv)
    pltpu.sync_copy(dst_hbm.at[w], dstv)


_agg_edge = functools.partial(
    pl.kernel,
    mesh=_mesh,
    out_type=jax.ShapeDtypeStruct((2, N_ACC, WP), jnp.float32),
    scratch_types=[
        pltpu.VMEM((CHE, CHUNK), jnp.int32),
        pltpu.VMEM((CHE, CHUNK), jnp.int32),
        pltpu.VMEM((CHUNK, WP), jnp.float32),
        pltpu.VMEM_SHARED((N_ACC, WP), jnp.float32),
        pltpu.SemaphoreType.DMA,
    ],
)(_agg_common(CHE, _load_idx_edge))


def _mm_first_body(x_ref, w_ref, deg_ref, o_ref):
    m = jnp.dot(x_ref[...], w_ref[...], preferred_element_type=jnp.float32)
    dis = lax.rsqrt(deg_ref[0, :, 0:1] + deg_ref[1, :, 0:1] + 1.0)
    o_ref[...] = dis * m


def _mm_first(x, w, degc):
    fi, fo = w.shape
    return pl.pallas_call(
        _mm_first_body,
        grid=(N // BM,),
        in_specs=[
            pl.BlockSpec((BM, fi), lambda i: (i, 0)),
            pl.BlockSpec((fi, fo), lambda i: (0, 0)),
            pl.BlockSpec((2, BM, 16), lambda i: (0, i, 0)),
        ],
        out_specs=pl.BlockSpec((BM, fo), lambda i: (i, 0)),
        out_shape=jax.ShapeDtypeStruct((N, fo), jnp.float32),
    )(x, w, degc)


def _bn_fold(st_ref, g_ref, be_ref):
    st = st_ref[...]
    mu = st[0:1] / N
    var = st[1:2] / N - mu * mu
    sv = g_ref[...] * lax.rsqrt(var + EPS)
    tv = be_ref[...] - mu * sv
    return sv, tv


def _mm_bn_body(split, r_ref, st_ref, g_ref, be_ref, w_ref, deg_ref, o_ref):
    sv, tv = _bn_fold(st_ref, g_ref, be_ref)
    a = r_ref[...] * sv + tv
    m = jnp.dot(a, w_ref[...], preferred_element_type=jnp.float32)
    dis = lax.rsqrt(deg_ref[0, :, 0:1] + deg_ref[1, :, 0:1] + 1.0)
    p = dis * m
    if split:
        o_ref[0] = p[:, :WP]
        o_ref[1] = p[:, WP:]
    else:
        o_ref[...] = p


def _mm_bn(r, st, g, be, w, degc, split):
    fi, fo = w.shape
    if split:
        out_spec = pl.BlockSpec((2, BM, WP), lambda i: (0, i, 0))
        out_shape = jax.ShapeDtypeStruct((2, N, WP), jnp.float32)
    else:
        out_spec = pl.BlockSpec((BM, fo), lambda i: (i, 0))
        out_shape = jax.ShapeDtypeStruct((N, fo), jnp.float32)
    return pl.pallas_call(
        functools.partial(_mm_bn_body, split),
        grid=(N // BM,),
        in_specs=[
            pl.BlockSpec((BM, fi), lambda i: (i, 0)),
            pl.BlockSpec((2, fi), lambda i: (0, 0)),
            pl.BlockSpec((1, fi), lambda i: (0, 0)),
            pl.BlockSpec((1, fi), lambda i: (0, 0)),
            pl.BlockSpec((fi, fo), lambda i: (0, 0)),
            pl.BlockSpec((2, BM, 16), lambda i: (0, i, 0)),
        ],
        out_specs=out_spec,
        out_shape=out_shape,
    )(r, st, g, be, w, degc)


def _epi_finish(ag, b_ref, deg_ref, r_ref, st_ref):
    dis = lax.rsqrt(deg_ref[0, :, 0:1] + deg_ref[1, :, 0:1] + 1.0)
    z = dis * ag + b_ref[...]
    r = jnp.maximum(z, 0.0)
    r_ref[...] = r
    srow = jnp.sum(r, axis=0, keepdims=True)
    s2row = jnp.sum(r * r, axis=0, keepdims=True)
    stnew = jnp.concatenate([srow, s2row], axis=0)

    @pl.when(pl.program_id(0) == 0)
    def _():
        st_ref[...] = stnew

    @pl.when(pl.program_id(0) > 0)
    def _():
        st_ref[...] += stnew


def _epi_body(f, acc_ref, p_ref, b_ref, deg_ref, r_ref, st_ref):
    ag = (acc_ref[0] + acc_ref[1] + p_ref[...])[:, :f]
    _epi_finish(ag, b_ref, deg_ref, r_ref, st_ref)


def _epi(acc, p, b, degc, f):
    return pl.pallas_call(
        functools.partial(_epi_body, f),
        grid=(N // BM,),
        in_specs=[
            pl.BlockSpec((2, BM, WP), lambda i: (0, i, 0)),
            pl.BlockSpec((BM, WP), lambda i: (i, 0)),
            pl.BlockSpec((1, f), lambda i: (0, 0)),
            pl.BlockSpec((2, BM, 16), lambda i: (0, i, 0)),
        ],
        out_specs=[
            pl.BlockSpec((BM, f), lambda i: (i, 0)),
            pl.BlockSpec((2, f), lambda i: (0, 0)),
        ],
        out_shape=[
            jax.ShapeDtypeStruct((N, f), jnp.float32),
            jax.ShapeDtypeStruct((2, f), jnp.float32),
        ],
    )(acc, p, b, degc)


def _epi2_body(acc_a_ref, acc_b_ref, p_ref, b_ref, deg_ref, r_ref, st_ref):
    ag = jnp.concatenate(
        [acc_a_ref[0] + acc_a_ref[1] + p_ref[0],
         acc_b_ref[0] + acc_b_ref[1] + p_ref[1]], axis=1)
    _epi_finish(ag, b_ref, deg_ref, r_ref, st_ref)


def _epi2(acc_a, acc_b, p, b, degc):
    f = 2 * WP
    return pl.pallas_call(
        _epi2_body,
        grid=(N // BM,),
        in_specs=[
            pl.BlockSpec((2, BM, WP), lambda i: (0, i, 0)),
            pl.BlockSpec((2, BM, WP), lambda i: (0, i, 0)),
            pl.BlockSpec((2, BM, WP), lambda i: (0, i, 0)),
            pl.BlockSpec((1, f), lambda i: (0, 0)),
            pl.BlockSpec((2, BM, 16), lambda i: (0, i, 0)),
        ],
        out_specs=[
            pl.BlockSpec((BM, f), lambda i: (i, 0)),
            pl.BlockSpec((2, f), lambda i: (0, 0)),
        ],
        out_shape=[
            jax.ShapeDtypeStruct((N, f), jnp.float32),
            jax.ShapeDtypeStruct((2, f), jnp.float32),
        ],
    )(acc_a, acc_b, p, b, degc)


def _cls_body(r_ref, st_ref, g_ref, be_ref, w_ref, b_ref, o_ref):
    sv, tv = _bn_fold(st_ref, g_ref, be_ref)
    a = r_ref[...] * sv + tv
    o_ref[...] = jnp.dot(a, w_ref[...],
                         preferred_element_type=jnp.float32) + b_ref[...]


def _cls(r, st, g, be, w, b):
    fi, nc = w.shape
    return pl.pallas_call(
        _cls_body,
        grid=(N // BM,),
        in_specs=[
            pl.BlockSpec((BM, fi), lambda i: (i, 0)),
            pl.BlockSpec((2, fi), lambda i: (0, 0)),
            pl.BlockSpec((1, fi), lambda i: (0, 0)),
            pl.BlockSpec((1, fi), lambda i: (0, 0)),
            pl.BlockSpec((fi, nc), lambda i: (0, 0)),
            pl.BlockSpec((1, nc), lambda i: (0, 0)),
        ],
        out_specs=pl.BlockSpec((BM, nc), lambda i: (i, 0)),
        out_shape=jax.ShapeDtypeStruct((N, nc), jnp.float32),
    )(r, st, g, be, w, b)


def kernel(x, edge_index, W1, b1, g1, be1, W2, b2, g2, be2, W3, b3, g3, be3,
           W4, b4, g4, be4, Wc, bc):
    src, dst = edge_index[0], edge_index[1]
    pad = E_PAD - E
    srcp = jnp.concatenate([src, jnp.zeros((pad,), jnp.int32)])
    dstp = jnp.concatenate([dst, jnp.full((pad,), N, jnp.int32)])
    src_e = srcp.reshape(2 * NS, CHE, CHUNK)
    dst_e = dstp.reshape(2 * NS, CHE, CHUNK)

    ones = jnp.ones((N, WP), jnp.float32)
    degc = _agg_edge(ones, src_e, dst_e)[:, :, :16]

    def layer_edge(p, bias, f):
        acc = _agg_edge(p, src_e, dst_e)
        return _epi(acc, p, bias.reshape(1, -1), degc, f)

    p1 = _mm_first(x, W1, degc)
    r1, st1 = layer_edge(p1, b1, 128)
    p2 = _mm_bn(r1, st1, g1.reshape(1, -1), be1.reshape(1, -1), W2, degc,
                split=True)
    acc2a = _agg_edge(p2[0], src_e, dst_e)
    acc2b = _agg_edge(p2[1], src_e, dst_e)
    r2, st2 = _epi2(acc2a, acc2b, p2, b2.reshape(1, -1), degc)
    p3 = _mm_bn(r2, st2, g2.reshape(1, -1), be2.reshape(1, -1), W3, degc,
                split=False)
    r3, st3 = layer_edge(p3, b3, 128)
    w4p = jnp.pad(W4, ((0, 0), (0, WP - W4.shape[1])))
    p4 = _mm_bn(r3, st3, g3.reshape(1, -1), be3.reshape(1, -1), w4p, degc,
                split=False)
    r4, st4 = layer_edge(p4, b4, 64)
    return _cls(r4, st4, g4.reshape(1, -1), be4.reshape(1, -1),
                Wc, bc.reshape(1, -1))
